# baseline (speedup 1.0000x reference)
import jax
import jax.numpy as jnp
from jax.experimental import pallas as pl
from jax.experimental.pallas import tpu as pltpu

_B = 256


def _net_kernel(xc_ref, a1_ref, b1_ref, a2_ref, b2_ref, w1_ref, bf1_ref,
                wc_ref, bc_ref, o_ref, p1_ref):
    f32 = jnp.float32
    bf16 = jnp.bfloat16
    b = o_ref.shape[0]

    p1_ref[0:b, 0:512] = jnp.zeros((b, 512), bf16)
    p1_ref[13 * b:14 * b, 1024:1536] = jnp.zeros((b, 512), bf16)
    for k in range(3):
        p1_ref[:, 512 * k:512 * k + 32] = jnp.zeros((14 * b, 32), bf16)
        p1_ref[:, 512 * k + 480:512 * k + 512] = jnp.zeros((14 * b, 32), bf16)

    a1 = a1_ref[...]
    b1 = b1_ref[...]

    for h2 in range(14):
        o = jnp.dot(xc_ref[h2, 0], a1, preferred_element_type=f32)
        m = jnp.maximum(o[0:b], o[b:2 * b])
        m = jnp.maximum(m + b1, 0.0).astype(bf16)
        p = jnp.concatenate(
            [jnp.maximum(m[:, 64 * i:64 * i + 32], m[:, 64 * i + 32:64 * i + 64])
             for i in range(14)], axis=1)
        r = h2 + 1
        for k in range(3):
            q = r - k
            if 0 <= q <= 13:
                p1_ref[q * b:(q + 1) * b, 512 * k + 32:512 * k + 480] = p

    b2 = b2_ref[...]
    acc = jnp.zeros((b, 128), f32)
    for h in range(7):
        r0 = 2 * h
        lhs = p1_ref[r0 * b:(r0 + 2) * b, :]
        o2 = jnp.dot(lhs, a2_ref[...], preferred_element_type=f32)
        m = jnp.maximum(o2[0:b], o2[b:2 * b])
        m = jnp.maximum(m + b2, 0.0)
        p = jnp.concatenate(
            [jnp.maximum(m[:, 128 * i:128 * i + 64], m[:, 128 * i + 64:128 * i + 128])
             for i in range(7)], axis=1)
        acc = acc + jnp.dot(p, w1_ref[h], preferred_element_type=f32)

    h1 = jnp.maximum(acc + bf1_ref[...], 0.0)
    o_ref[...] = jnp.dot(h1, wc_ref[...], preferred_element_type=f32) + bc_ref[...]


def kernel(a1, b1, a2, b2, w_fc1, b_fc1, w_cls, b_cls, x_nchw):
    n = x_nchw.shape[0]
    c = w_cls.shape[1]
    bf16 = jnp.bfloat16
    np_ = pl.cdiv(n, _B) * _B
    nb = np_ // _B
    x = x_nchw[:, 0]
    xpad = jnp.pad(x, ((0, np_ - n), (1, 1), (1, 1)))
    rows = jnp.stack([xpad[:, r:r + 3, :].reshape(np_, 90)
                      for r in range(28)])
    rows = rows.reshape(28, nb, _B, 90)
    xc = jnp.concatenate([rows[0::2], rows[1::2]], axis=2)
    xc = xc.astype(bf16)
    a1c = a1.reshape(90, 896).astype(bf16)
    a2c = a2.reshape(1536, 896).astype(bf16)
    w1r = w_fc1.reshape(7, 448, 128)
    out = pl.pallas_call(
        _net_kernel,
        out_shape=jax.ShapeDtypeStruct((np_, c), jnp.float32),
        grid_spec=pltpu.PrefetchScalarGridSpec(
            num_scalar_prefetch=0,
            grid=(nb,),
            in_specs=[
                pl.BlockSpec((14, 1, 2 * _B, 90), lambda i: (0, i, 0, 0)),
                pl.BlockSpec((90, 896), lambda i: (0, 0)),
                pl.BlockSpec((1, 896), lambda i: (0, 0)),
                pl.BlockSpec((1536, 896), lambda i: (0, 0)),
                pl.BlockSpec((1, 896), lambda i: (0, 0)),
                pl.BlockSpec((7, 448, 128), lambda i: (0, 0, 0)),
                pl.BlockSpec((1, 128), lambda i: (0, 0)),
                pl.BlockSpec((128, c), lambda i: (0, 0)),
                pl.BlockSpec((1, c), lambda i: (0, 0)),
            ],
            out_specs=pl.BlockSpec((_B, c), lambda i: (i, 0)),
            scratch_shapes=[pltpu.VMEM((14 * _B, 1536), bf16)],
        ),
        compiler_params=pltpu.CompilerParams(
            dimension_semantics=("parallel",),
            vmem_limit_bytes=48 * 1024 * 1024,
        ),
    )(xc, a1c, b1, a2c, b2, w1r, b_fc1, w_cls, b_cls)
    return out[:n] if np_ != n else out


# speedup vs baseline: 1.0138x; 1.0138x over previous
import jax
import jax.numpy as jnp
from jax.experimental import pallas as pl
from jax.experimental.pallas import tpu as pltpu

_B = 256


def _net_kernel(xc_ref, a1_ref, b1_ref, a2_ref, b2_ref, w1_ref, bf1_ref,
                wc_ref, bc_ref, o_ref, p1_ref):
    f32 = jnp.float32
    bf16 = jnp.bfloat16
    b = o_ref.shape[0]

    p1_ref[0:b, :] = jnp.zeros((b, 512), bf16)
    p1_ref[15 * b:16 * b, :] = jnp.zeros((b, 512), bf16)
    p1_ref[:, 0:32] = jnp.zeros((16 * b, 32), bf16)
    p1_ref[:, 480:512] = jnp.zeros((16 * b, 32), bf16)

    a1 = a1_ref[...]
    b1 = b1_ref[...]

    for h2 in range(14):
        o = jnp.dot(xc_ref[h2, 0], a1, preferred_element_type=f32)
        m = jnp.maximum(o[0:b], o[b:2 * b])
        m = jnp.maximum(m + b1, 0.0).astype(bf16)
        p = jnp.concatenate(
            [jnp.maximum(m[:, 64 * i:64 * i + 32], m[:, 64 * i + 32:64 * i + 64])
             for i in range(14)], axis=1)
        p1_ref[(h2 + 1) * b:(h2 + 2) * b, 32:480] = p

    b2 = b2_ref[...]
    acc = jnp.zeros((b, 128), f32)
    for h in range(7):
        r0 = 2 * h
        o2 = jnp.dot(p1_ref[r0 * b:(r0 + 2) * b, :], a2_ref[0],
                     preferred_element_type=f32)
        o2 = o2 + jnp.dot(p1_ref[(r0 + 1) * b:(r0 + 3) * b, :], a2_ref[1],
                          preferred_element_type=f32)
        o2 = o2 + jnp.dot(p1_ref[(r0 + 2) * b:(r0 + 4) * b, :], a2_ref[2],
                          preferred_element_type=f32)
        m = jnp.maximum(o2[0:b], o2[b:2 * b])
        m = jnp.maximum(m + b2, 0.0)
        p = jnp.concatenate(
            [jnp.maximum(m[:, 128 * i:128 * i + 64], m[:, 128 * i + 64:128 * i + 128])
             for i in range(7)], axis=1)
        acc = acc + jnp.dot(p, w1_ref[h], preferred_element_type=f32)

    h1 = jnp.maximum(acc + bf1_ref[...], 0.0)
    o_ref[...] = jnp.dot(h1, wc_ref[...], preferred_element_type=f32) + bc_ref[...]


def kernel(a1, b1, a2, b2, w_fc1, b_fc1, w_cls, b_cls, x_nchw):
    n = x_nchw.shape[0]
    c = w_cls.shape[1]
    bf16 = jnp.bfloat16
    np_ = pl.cdiv(n, _B) * _B
    nb = np_ // _B
    x = x_nchw[:, 0]
    xpad = jnp.pad(x, ((0, np_ - n), (1, 1), (1, 1)))
    rows = jnp.stack([xpad[:, r:r + 3, :].reshape(np_, 90)
                      for r in range(28)])
    rows = rows.reshape(28, nb, _B, 90)
    xc = jnp.concatenate([rows[0::2], rows[1::2]], axis=2)
    xc = xc.astype(bf16)
    a1c = a1.reshape(90, 896).astype(bf16)
    a2c = a2.astype(bf16)
    w1r = w_fc1.reshape(7, 448, 128)
    out = pl.pallas_call(
        _net_kernel,
        out_shape=jax.ShapeDtypeStruct((np_, c), jnp.float32),
        grid_spec=pltpu.PrefetchScalarGridSpec(
            num_scalar_prefetch=0,
            grid=(nb,),
            in_specs=[
                pl.BlockSpec((14, 1, 2 * _B, 90), lambda i: (0, i, 0, 0)),
                pl.BlockSpec((90, 896), lambda i: (0, 0)),
                pl.BlockSpec((1, 896), lambda i: (0, 0)),
                pl.BlockSpec((3, 512, 896), lambda i: (0, 0, 0)),
                pl.BlockSpec((1, 896), lambda i: (0, 0)),
                pl.BlockSpec((7, 448, 128), lambda i: (0, 0, 0)),
                pl.BlockSpec((1, 128), lambda i: (0, 0)),
                pl.BlockSpec((128, c), lambda i: (0, 0)),
                pl.BlockSpec((1, c), lambda i: (0, 0)),
            ],
            out_specs=pl.BlockSpec((_B, c), lambda i: (i, 0)),
            scratch_shapes=[pltpu.VMEM((16 * _B, 512), bf16)],
        ),
        compiler_params=pltpu.CompilerParams(
            dimension_semantics=("parallel",),
            vmem_limit_bytes=48 * 1024 * 1024,
        ),
    )(xc, a1c, b1, a2c, b2, w1r, b_fc1, w_cls, b_cls)
    return out[:n] if np_ != n else out


# speedup vs baseline: 1.4124x; 1.3931x over previous
import jax
import jax.numpy as jnp
from jax.experimental import pallas as pl
from jax.experimental.pallas import tpu as pltpu

_B = 256


def _net_kernel(x_ref, a1_ref, b1_ref, a2_ref, b2_ref, w1_ref, bf1_ref,
                wc_ref, bc_ref, o_ref, p1_ref):
    f32 = jnp.float32
    bf16 = jnp.bfloat16
    b = o_ref.shape[0]

    p1_ref[0:b, :] = jnp.zeros((b, 512), bf16)
    p1_ref[15 * b:16 * b, :] = jnp.zeros((b, 512), bf16)
    p1_ref[:, 0:32] = jnp.zeros((16 * b, 32), bf16)
    p1_ref[:, 480:512] = jnp.zeros((16 * b, 32), bf16)

    a1 = a1_ref[...]
    b1 = b1_ref[...]

    for h2 in range(14):
        r = 2 * h2
        xa = jnp.concatenate([x_ref[r], x_ref[r + 1], x_ref[r + 2]], axis=1)
        xb = jnp.concatenate([x_ref[r + 1], x_ref[r + 2], x_ref[r + 3]], axis=1)
        xp = jnp.concatenate([xa, xb], axis=0)
        o = jnp.dot(xp, a1, preferred_element_type=f32)
        m = jnp.maximum(o[0:b], o[b:2 * b])
        m = jnp.maximum(m + b1, 0.0).astype(bf16)
        p = jnp.concatenate(
            [jnp.maximum(m[:, 64 * i:64 * i + 32], m[:, 64 * i + 32:64 * i + 64])
             for i in range(14)], axis=1)
        p1_ref[(h2 + 1) * b:(h2 + 2) * b, 32:480] = p

    b2 = b2_ref[...]
    acc = jnp.zeros((b, 128), f32)
    for h in range(7):
        r0 = 2 * h
        o2 = jnp.dot(p1_ref[r0 * b:(r0 + 2) * b, :], a2_ref[0],
                     preferred_element_type=f32)
        o2 = o2 + jnp.dot(p1_ref[(r0 + 1) * b:(r0 + 3) * b, :], a2_ref[1],
                          preferred_element_type=f32)
        o2 = o2 + jnp.dot(p1_ref[(r0 + 2) * b:(r0 + 4) * b, :], a2_ref[2],
                          preferred_element_type=f32)
        m = jnp.maximum(o2[0:b], o2[b:2 * b])
        m = jnp.maximum(m + b2, 0.0)
        p = jnp.concatenate(
            [jnp.maximum(m[:, 128 * i:128 * i + 64], m[:, 128 * i + 64:128 * i + 128])
             for i in range(7)], axis=1)
        acc = acc + jnp.dot(p, w1_ref[h], preferred_element_type=f32)

    h1 = jnp.maximum(acc + bf1_ref[...], 0.0)
    o_ref[...] = jnp.dot(h1, wc_ref[...], preferred_element_type=f32) + bc_ref[...]


def kernel(a1, b1, a2, b2, w_fc1, b_fc1, w_cls, b_cls, x_nchw):
    n = x_nchw.shape[0]
    c = w_cls.shape[1]
    bf16 = jnp.bfloat16
    np_ = pl.cdiv(n, _B) * _B
    x = x_nchw[:, 0]
    xt = jnp.pad(x.astype(bf16), ((0, np_ - n), (1, 1), (1, 1))).transpose(1, 0, 2)
    a1c = a1.reshape(90, 896).astype(bf16)
    a2c = a2.astype(bf16)
    w1r = w_fc1.reshape(7, 448, 128)
    out = pl.pallas_call(
        _net_kernel,
        out_shape=jax.ShapeDtypeStruct((np_, c), jnp.float32),
        grid_spec=pltpu.PrefetchScalarGridSpec(
            num_scalar_prefetch=0,
            grid=(np_ // _B,),
            in_specs=[
                pl.BlockSpec((30, _B, 30), lambda i: (0, i, 0)),
                pl.BlockSpec((90, 896), lambda i: (0, 0)),
                pl.BlockSpec((1, 896), lambda i: (0, 0)),
                pl.BlockSpec((3, 512, 896), lambda i: (0, 0, 0)),
                pl.BlockSpec((1, 896), lambda i: (0, 0)),
                pl.BlockSpec((7, 448, 128), lambda i: (0, 0, 0)),
                pl.BlockSpec((1, 128), lambda i: (0, 0)),
                pl.BlockSpec((128, c), lambda i: (0, 0)),
                pl.BlockSpec((1, c), lambda i: (0, 0)),
            ],
            out_specs=pl.BlockSpec((_B, c), lambda i: (i, 0)),
            scratch_shapes=[pltpu.VMEM((16 * _B, 512), bf16)],
        ),
        compiler_params=pltpu.CompilerParams(
            dimension_semantics=("parallel",),
            vmem_limit_bytes=48 * 1024 * 1024,
        ),
    )(xt, a1c, b1, a2c, b2, w1r, b_fc1, w_cls, b_cls)
    return out[:n] if np_ != n else out
